# baseline (speedup 1.0000x reference)
import functools

import jax
import jax.numpy as jnp
from jax.experimental import pallas as pl
from jax.experimental.pallas import tpu as pltpu
from jax.experimental.pallas import tpu_sc as plsc


_W = 128


def _sc_gather(c_table, c_idx, m_table, m_idx):
    nc = c_idx.shape[1]
    dim = c_table.shape[1]
    nc_main = (nc // _W) * _W
    nc_tail = nc - nc_main
    nm_rows, wm = m_idx.shape
    nm = nm_rows * wm
    mesh = plsc.VectorSubcoreMesh(core_axis_name="c", subcore_axis_name="s")

    @pl.kernel(
        out_type=(
            jax.ShapeDtypeStruct((nc, dim), c_table.dtype),
            jax.ShapeDtypeStruct((nm, dim), m_table.dtype),
        ),
        mesh=mesh,
    )
    def gather_kernel(c_hbm, ci_hbm, m_hbm, mi_hbm, o_dia, o_med):
        core = jax.lax.axis_index("c")

        @pl.when(core == 0)
        def _():
            def body_c(i_vmem, o_vmem):
                pltpu.sync_copy(c_hbm.at[i_vmem.at[0]], o_vmem)

            pltpu.emit_pipeline(
                body_c,
                grid=(nc_main // _W,),
                in_specs=[pl.BlockSpec((1, _W), lambda i: (0, i))],
                out_specs=[pl.BlockSpec((_W, dim), lambda i: (i, 0))],
                core_axis_name="s",
                dimension_semantics=(pltpu.PARALLEL,),
            )(ci_hbm, o_dia)

            if nc_tail:
                pltpu.emit_pipeline(
                    body_c,
                    grid=(1,),
                    in_specs=[pl.BlockSpec((1, nc_tail), lambda i: (0, 0))],
                    out_specs=[pl.BlockSpec((nc_tail, dim),
                                            lambda i: (0, 0))],
                    core_axis_name="s",
                    dimension_semantics=(pltpu.PARALLEL,),
                )(ci_hbm.at[:, nc_main:], o_dia.at[nc_main:, :])

        @pl.when(core == 1)
        def _():
            def body_m(i_vmem, o_vmem):
                pltpu.sync_copy(m_hbm.at[i_vmem.at[0]], o_vmem)

            pltpu.emit_pipeline(
                body_m,
                grid=(nm_rows,),
                in_specs=[pl.BlockSpec((1, wm), lambda i: (i, 0))],
                out_specs=[pl.BlockSpec((wm, dim), lambda i: (i, 0))],
                core_axis_name="s",
                dimension_semantics=(pltpu.PARALLEL,),
            )(mi_hbm, o_med)

    return gather_kernel(c_table, c_idx, m_table, m_idx)


def _tc_body(nc, nm, dia_ref, med_ref, hat, w1, b1, w2, att2, b2, wl,
             o1, o2):
    f32 = jnp.float32
    c = w2.shape[1]
    dia = dia_ref[...][:nc]
    med = med_ref[...][:nm]
    xd = jnp.dot(dia, w2[...], preferred_element_type=f32)
    xm = jnp.dot(med, w2[...], preferred_element_type=f32)
    he = jnp.dot(hat[...], w2[...], preferred_element_type=f32)

    att = att2[...]
    an = att[:c][None, :]
    ae = att[c:][None, :]
    b1v = b1[...][None, :]
    b2v = b2[...][None, :]
    wl_t = wl[...][:c]
    wl_b = wl[...][c:]
    v = jnp.sum(he * ae, axis=1, keepdims=True)
    ud = jnp.sum(xd * an, axis=1, keepdims=True)
    um = jnp.sum(xm * an, axis=1)

    lrelu = lambda x: jnp.where(x >= 0, x, 0.2 * x)
    a_dis = lrelu(ud + v)
    amat = lrelu(v + um[None, :])
    a_max = jnp.maximum(jnp.max(amat, axis=1, keepdims=True), a_dis)
    emat = jnp.exp(amat - a_max)
    p = jnp.exp(a_dis - a_max)
    ssum = jnp.sum(emat, axis=1, keepdims=True)
    denom = p + ssum + 1e-16
    g = jnp.dot(emat, xm, preferred_element_type=f32)
    ef = (p * xd + g) / denom * (1.0 / (nm + 1))
    sum1 = jnp.sum((p / denom) * ef, axis=0)[None, :]
    sum2 = jnp.sum((ssum / denom) * ef, axis=0)[None, :]

    sum_dia = jnp.sum(dia, axis=0)[None, :]
    sum_med = jnp.sum(med, axis=0)[None, :]
    t1 = jnp.dot(sum_dia, w1[...], preferred_element_type=f32) + nc * b1v
    t2 = jnp.dot(sum_med, w1[...], preferred_element_type=f32) + nm * b1v

    r1 = sum1 + nc * b2v
    r2 = sum2 * (1.0 / nc) + nm * b2v
    o1[...] = (jnp.dot(r1, wl_t, preferred_element_type=f32)
               + jnp.dot(t1, wl_b, preferred_element_type=f32))
    o2[...] = (jnp.dot(r2, wl_t, preferred_element_type=f32)
               + jnp.dot(t2, wl_b, preferred_element_type=f32))


def kernel(c_it, medicine_it, c_embeddings, m_embeddings, W1, b1, W2, att2,
           b2, Wl, hyperedge_attr):
    nc = c_it.shape[0]
    nm = medicine_it.shape[0]
    c = W2.shape[1]

    wm = 32 if nm % 32 == 0 else 1
    ci = c_it.astype(jnp.int32).reshape(1, nc)
    mi = medicine_it.astype(jnp.int32).reshape(nm // wm, wm)
    dia, med = _sc_gather(c_embeddings, ci, m_embeddings, mi)

    i1, i2 = pl.pallas_call(
        functools.partial(_tc_body, nc, nm),
        out_shape=(
            jax.ShapeDtypeStruct((1, c), jnp.float32),
            jax.ShapeDtypeStruct((1, c), jnp.float32),
        ),
    )(dia, med, hyperedge_attr, W1, b1, W2, att2, b2, Wl)

    return i1.reshape(1, 1, c), i2.reshape(1, 1, c)


# speedup vs baseline: 1.0632x; 1.0632x over previous
import functools

import jax
import jax.numpy as jnp
from jax.experimental import pallas as pl
from jax.experimental.pallas import tpu as pltpu
from jax.experimental.pallas import tpu_sc as plsc


_W = 128


def _sc_gather(c_table, c_idx, m_table, m_idx):
    nc_rows, wc = c_idx.shape
    nc = nc_rows * wc
    nm_rows, wm = m_idx.shape
    nm = nm_rows * wm
    dim = c_table.shape[1]
    mesh = plsc.VectorSubcoreMesh(core_axis_name="c", subcore_axis_name="s")

    @pl.kernel(
        out_type=(
            jax.ShapeDtypeStruct((nc, dim), c_table.dtype),
            jax.ShapeDtypeStruct((nm, dim), m_table.dtype),
        ),
        mesh=mesh,
    )
    def gather_kernel(c_hbm, ci_hbm, m_hbm, mi_hbm, o_dia, o_med):
        core = jax.lax.axis_index("c")

        @pl.when(core == 0)
        def _():
            def body_c(i_vmem, o_vmem):
                pltpu.sync_copy(c_hbm.at[i_vmem.at[0]], o_vmem)

            pltpu.emit_pipeline(
                body_c,
                grid=(nc_rows,),
                in_specs=[pl.BlockSpec((1, wc), lambda i: (i, 0))],
                out_specs=[pl.BlockSpec((wc, dim), lambda i: (i, 0))],
                core_axis_name="s",
                dimension_semantics=(pltpu.PARALLEL,),
            )(ci_hbm, o_dia)

        @pl.when(core == 1)
        def _():
            def body_m(i_vmem, o_vmem):
                pltpu.sync_copy(m_hbm.at[i_vmem.at[0]], o_vmem)

            pltpu.emit_pipeline(
                body_m,
                grid=(nm_rows,),
                in_specs=[pl.BlockSpec((1, wm), lambda i: (i, 0))],
                out_specs=[pl.BlockSpec((wm, dim), lambda i: (i, 0))],
                core_axis_name="s",
                dimension_semantics=(pltpu.PARALLEL,),
            )(mi_hbm, o_med)

    return gather_kernel(c_table, c_idx, m_table, m_idx)


def _tc_body(nc, nm, dia_ref, med_ref, hat, w1, b1, w2, att2, b2, wl,
             o1, o2):
    f32 = jnp.float32
    c = w2.shape[1]
    dia = dia_ref[...][:nc]
    med = med_ref[...][:nm]
    xd = jnp.dot(dia, w2[...], preferred_element_type=f32)
    xm = jnp.dot(med, w2[...], preferred_element_type=f32)
    he = jnp.dot(hat[...], w2[...], preferred_element_type=f32)

    att = att2[...]
    an = att[:c][None, :]
    ae = att[c:][None, :]
    b1v = b1[...][None, :]
    b2v = b2[...][None, :]
    wl_t = wl[...][:c]
    wl_b = wl[...][c:]
    v = jnp.sum(he * ae, axis=1, keepdims=True)
    ud = jnp.sum(xd * an, axis=1, keepdims=True)
    um = jnp.sum(xm * an, axis=1)

    lrelu = lambda x: jnp.where(x >= 0, x, 0.2 * x)
    a_dis = lrelu(ud + v)
    amat = lrelu(v + um[None, :])
    a_max = jnp.maximum(jnp.max(amat, axis=1, keepdims=True), a_dis)
    emat = jnp.exp(amat - a_max)
    p = jnp.exp(a_dis - a_max)
    ssum = jnp.sum(emat, axis=1, keepdims=True)
    denom = p + ssum + 1e-16
    g = jnp.dot(emat, xm, preferred_element_type=f32)
    ef = (p * xd + g) / denom * (1.0 / (nm + 1))
    sum1 = jnp.sum((p / denom) * ef, axis=0)[None, :]
    sum2 = jnp.sum((ssum / denom) * ef, axis=0)[None, :]

    sum_dia = jnp.sum(dia, axis=0)[None, :]
    sum_med = jnp.sum(med, axis=0)[None, :]
    t1 = jnp.dot(sum_dia, w1[...], preferred_element_type=f32) + nc * b1v
    t2 = jnp.dot(sum_med, w1[...], preferred_element_type=f32) + nm * b1v

    r1 = sum1 + nc * b2v
    r2 = sum2 * (1.0 / nc) + nm * b2v
    o1[...] = (jnp.dot(r1, wl_t, preferred_element_type=f32)
               + jnp.dot(t1, wl_b, preferred_element_type=f32))
    o2[...] = (jnp.dot(r2, wl_t, preferred_element_type=f32)
               + jnp.dot(t2, wl_b, preferred_element_type=f32))


def kernel(c_it, medicine_it, c_embeddings, m_embeddings, W1, b1, W2, att2,
           b2, Wl, hyperedge_attr):
    nc = c_it.shape[0]
    nm = medicine_it.shape[0]
    c = W2.shape[1]

    nsub = 16
    wc = (-(-nc // nsub) + 7) // 8 * 8
    ci = jnp.zeros((nsub, wc), jnp.int32).reshape(-1).at[:nc].set(
        c_it.astype(jnp.int32)).reshape(nsub, wc)
    wm = nm // nsub if nm % nsub == 0 else 1
    mi = medicine_it.astype(jnp.int32).reshape(nm // wm, wm)
    dia, med = _sc_gather(c_embeddings, ci, m_embeddings, mi)

    i1, i2 = pl.pallas_call(
        functools.partial(_tc_body, nc, nm),
        out_shape=(
            jax.ShapeDtypeStruct((1, c), jnp.float32),
            jax.ShapeDtypeStruct((1, c), jnp.float32),
        ),
    )(dia, med, hyperedge_attr, W1, b1, W2, att2, b2, Wl)

    return i1.reshape(1, 1, c), i2.reshape(1, 1, c)


# speedup vs baseline: 1.0647x; 1.0015x over previous
import functools

import jax
import jax.numpy as jnp
from jax.experimental import pallas as pl
from jax.experimental.pallas import tpu as pltpu
from jax.experimental.pallas import tpu_sc as plsc


def _sc_gather(c_table, c_idx, m_table, m_idx):
    wc = wm = 64
    nc = c_idx.shape[0]
    nm = m_idx.shape[0]
    nc_rows = nc // wc
    nm_rows = nm // wm
    dim = c_table.shape[1]
    mesh = plsc.VectorSubcoreMesh(core_axis_name="c", subcore_axis_name="s")

    @pl.kernel(
        out_type=(
            jax.ShapeDtypeStruct((nc, dim), c_table.dtype),
            jax.ShapeDtypeStruct((nm, dim), m_table.dtype),
        ),
        mesh=mesh,
    )
    def gather_kernel(c_hbm, ci_hbm, m_hbm, mi_hbm, o_dia, o_med):
        core = jax.lax.axis_index("c")

        @pl.when(core == 0)
        def _():
            def body_c(i_vmem, o_vmem):
                pltpu.sync_copy(c_hbm.at[i_vmem], o_vmem)

            pltpu.emit_pipeline(
                body_c,
                grid=(nc_rows,),
                in_specs=[pl.BlockSpec((wc,), lambda i: (i,))],
                out_specs=[pl.BlockSpec((wc, dim), lambda i: (i, 0))],
                core_axis_name="s",
                dimension_semantics=(pltpu.PARALLEL,),
            )(ci_hbm, o_dia)

        @pl.when(core == 1)
        def _():
            def body_m(i_vmem, o_vmem):
                pltpu.sync_copy(m_hbm.at[i_vmem], o_vmem)

            pltpu.emit_pipeline(
                body_m,
                grid=(nm_rows,),
                in_specs=[pl.BlockSpec((wm,), lambda i: (i,))],
                out_specs=[pl.BlockSpec((wm, dim), lambda i: (i, 0))],
                core_axis_name="s",
                dimension_semantics=(pltpu.PARALLEL,),
            )(mi_hbm, o_med)

    return gather_kernel(c_table, c_idx, m_table, m_idx)


def _tc_body(nc, nm, dia_ref, med_ref, hat, w1, b1, w2, att2, b2, wl,
             o1, o2):
    f32 = jnp.float32
    c = w2.shape[1]
    dia = dia_ref[...][:nc]
    med = med_ref[...][:nm]
    xd = jnp.dot(dia, w2[...], preferred_element_type=f32)
    xm = jnp.dot(med, w2[...], preferred_element_type=f32)
    he = jnp.dot(hat[...], w2[...], preferred_element_type=f32)

    att = att2[...]
    an = att[:c][None, :]
    ae = att[c:][None, :]
    b1v = b1[...][None, :]
    b2v = b2[...][None, :]
    wl_t = wl[...][:c]
    wl_b = wl[...][c:]
    v = jnp.sum(he * ae, axis=1, keepdims=True)
    ud = jnp.sum(xd * an, axis=1, keepdims=True)
    um = jnp.sum(xm * an, axis=1)

    lrelu = lambda x: jnp.where(x >= 0, x, 0.2 * x)
    a_dis = lrelu(ud + v)
    amat = lrelu(v + um[None, :])
    a_max = jnp.maximum(jnp.max(amat, axis=1, keepdims=True), a_dis)
    emat = jnp.exp(amat - a_max)
    p = jnp.exp(a_dis - a_max)
    ssum = jnp.sum(emat, axis=1, keepdims=True)
    denom = p + ssum + 1e-16
    g = jnp.dot(emat, xm, preferred_element_type=f32)
    ef = (p * xd + g) / denom * (1.0 / (nm + 1))
    sum1 = jnp.sum((p / denom) * ef, axis=0)[None, :]
    sum2 = jnp.sum((ssum / denom) * ef, axis=0)[None, :]

    sum_dia = jnp.sum(dia, axis=0)[None, :]
    sum_med = jnp.sum(med, axis=0)[None, :]
    t1 = jnp.dot(sum_dia, w1[...], preferred_element_type=f32) + nc * b1v
    t2 = jnp.dot(sum_med, w1[...], preferred_element_type=f32) + nm * b1v

    r1 = sum1 + nc * b2v
    r2 = sum2 * (1.0 / nc) + nm * b2v
    o1[...] = (jnp.dot(r1, wl_t, preferred_element_type=f32)
               + jnp.dot(t1, wl_b, preferred_element_type=f32))
    o2[...] = (jnp.dot(r2, wl_t, preferred_element_type=f32)
               + jnp.dot(t2, wl_b, preferred_element_type=f32))


def kernel(c_it, medicine_it, c_embeddings, m_embeddings, W1, b1, W2, att2,
           b2, Wl, hyperedge_attr):
    nc = c_it.shape[0]
    nm = medicine_it.shape[0]
    c = W2.shape[1]

    w = 64
    nc_pad = -(-nc // w) * w
    ci = jnp.pad(c_it.astype(jnp.int32), (0, nc_pad - nc))
    mi = medicine_it.astype(jnp.int32)
    dia, med = _sc_gather(c_embeddings, ci, m_embeddings, mi)

    i1, i2 = pl.pallas_call(
        functools.partial(_tc_body, nc, nm),
        out_shape=(
            jax.ShapeDtypeStruct((1, c), jnp.float32),
            jax.ShapeDtypeStruct((1, c), jnp.float32),
        ),
    )(dia, med, hyperedge_attr, W1, b1, W2, att2, b2, Wl)

    return i1.reshape(1, 1, c), i2.reshape(1, 1, c)
